# Initial kernel scaffold; baseline (speedup 1.0000x reference)
import jax
import jax.numpy as jnp
from jax.experimental import pallas as pl


def kernel(x, pe_table):
    raise NotImplementedError("write your pallas kernel here")


# speedup vs baseline: 2.8486x; 2.8486x over previous
import jax
import jax.numpy as jnp
from jax.experimental import pallas as pl


def _add_body(x_ref, pe_ref, out_ref):
    out_ref[0] = x_ref[0] + pe_ref[...]


def kernel(x, pe_table):
    B, S, H = x.shape
    BS = 512
    grid = (S // BS, B)
    return pl.pallas_call(
        _add_body,
        grid=grid,
        in_specs=[
            pl.BlockSpec((1, BS, H), lambda s, b: (b, s, 0)),
            pl.BlockSpec((BS, H), lambda s, b: (s, 0)),
        ],
        out_specs=pl.BlockSpec((1, BS, H), lambda s, b: (b, s, 0)),
        out_shape=jax.ShapeDtypeStruct((B, S, H), x.dtype),
    )(x, pe_table)


# speedup vs baseline: 3.1798x; 1.1162x over previous
import jax
import jax.numpy as jnp
from jax.experimental import pallas as pl


def _add_body(x_ref, pe_ref, out_ref):
    out_ref[0] = x_ref[0] + pe_ref[...]


def kernel(x, pe_table):
    B, S, H = x.shape
    BS = 1024
    grid = (S // BS, B)
    return pl.pallas_call(
        _add_body,
        grid=grid,
        in_specs=[
            pl.BlockSpec((1, BS, H), lambda s, b: (b, s, 0)),
            pl.BlockSpec((BS, H), lambda s, b: (s, 0)),
        ],
        out_specs=pl.BlockSpec((1, BS, H), lambda s, b: (b, s, 0)),
        out_shape=jax.ShapeDtypeStruct((B, S, H), x.dtype),
    )(x, pe_table)


# speedup vs baseline: 3.3112x; 1.0413x over previous
import jax
import jax.numpy as jnp
from jax.experimental import pallas as pl


def _add_body(x_ref, pe_ref, out_ref):
    out_ref[0] = x_ref[0] + pe_ref[...]


def kernel(x, pe_table):
    B, S, H = x.shape
    BS = 2048
    grid = (S // BS, B)
    return pl.pallas_call(
        _add_body,
        grid=grid,
        in_specs=[
            pl.BlockSpec((1, BS, H), lambda s, b: (b, s, 0)),
            pl.BlockSpec((BS, H), lambda s, b: (s, 0)),
        ],
        out_specs=pl.BlockSpec((1, BS, H), lambda s, b: (b, s, 0)),
        out_shape=jax.ShapeDtypeStruct((B, S, H), x.dtype),
    )(x, pe_table)
